# Initial kernel scaffold; baseline (speedup 1.0000x reference)
import jax
import jax.numpy as jnp
from jax.experimental import pallas as pl


def kernel(x, params):
    raise NotImplementedError("write your pallas kernel here")


# speedup vs baseline: 437.1987x; 437.1987x over previous
import functools

import jax
import jax.numpy as jnp
import numpy as np
from jax.experimental import pallas as pl
from jax.experimental.pallas import tpu as pltpu

NS = 64
NA = 32
NN = 96
H = 4
EPS = 1e-5
NEG = -1e30

_EDGES = (
    (0, NS, 0, NS, 16, 0),
    (0, NS, NS, NA, 8, 1),
    (NS, NA, 0, NS, 8, 0),
    (NS, NA, NS, NA, 8, 1),
)

BB = 32


def _bn_rows(h, g, beta):
    n = h.shape[0]
    mu = jnp.sum(h, axis=0, keepdims=True) / n
    d = h - mu
    var = jnp.sum(d * d, axis=0, keepdims=True) / n
    return d / jnp.sqrt(var + EPS) * g + beta


def _topk_mask(sim, k):
    n_d = sim.shape[1]
    iota = jax.lax.broadcasted_iota(jnp.int32, sim.shape, 1)
    work = sim
    sel = jnp.zeros_like(sim)
    for _ in range(k):
        cm = jnp.max(work, axis=1, keepdims=True)
        cand = jnp.where(work == cm, iota, n_d)
        fi = jnp.min(cand, axis=1, keepdims=True)
        ch = iota == fi
        sel = jnp.maximum(sel, ch.astype(jnp.float32))
        work = jnp.where(ch, -3.0, work)
    return sel


def _graph_kernel(emb_ref, vecs_ref, wvs_ref, wva_ref,
                  vp_ref, mss_ref, msa_ref, mas_ref, maa_ref):
    emb = emb_ref[...]
    vecs = vecs_ref[...]
    v = _bn_rows(emb, vecs[0:1], vecs[1:2])
    vb = v.astype(jnp.bfloat16)
    hs = jnp.dot(vb[0:NS], wvs_ref[...].astype(jnp.bfloat16),
                 preferred_element_type=jnp.float32) + vecs[2:3]
    vps = _bn_rows(hs, vecs[3:4], vecs[4:5])
    ha = jnp.dot(vb[NS:NN], wva_ref[...].astype(jnp.bfloat16),
                 preferred_element_type=jnp.float32) + vecs[5:6]
    vpa = _bn_rows(ha, vecs[6:7], vecs[7:8])
    vp_ref[...] = jnp.concatenate([vps, vpa], axis=0)

    nrm = jnp.sqrt(jnp.sum(v * v, axis=1, keepdims=True)) + 1e-8
    outs = (mss_ref, msa_ref, mas_ref, maa_ref)
    for (s0, ns, d0, nd, k, _), oref in zip(_EDGES, outs):
        g = jax.lax.dot_general(vb[s0:s0 + ns], vb[d0:d0 + nd],
                                (((1,), (1,)), ((), ())),
                                preferred_element_type=jnp.float32)
        outer = jax.lax.dot_general(nrm[s0:s0 + ns], nrm[d0:d0 + nd],
                                    (((1,), (1,)), ((), ())),
                                    preferred_element_type=jnp.float32,
                                    precision=jax.lax.Precision.HIGHEST)
        oref[...] = _topk_mask(g / outer, k)


def _proj_kernel(x_ref, ws_ref, wa_ref, bv_ref, p_ref, st_ref):
    i = pl.program_id(0)
    xb = x_ref[...]
    L = xb.shape[2]
    D = ws_ref.shape[1]
    xs = xb[:, :NS, :].reshape(BB * NS, L)
    xa = xb[:, NS:, :].reshape(BB * NA, L)
    ps = jnp.dot(xs.astype(jnp.bfloat16), ws_ref[...].astype(jnp.bfloat16),
                 preferred_element_type=jnp.float32) + bv_ref[0:1]
    pa = jnp.dot(xa.astype(jnp.bfloat16), wa_ref[...].astype(jnp.bfloat16),
                 preferred_element_type=jnp.float32) + bv_ref[1:2]
    p_ref[...] = jnp.concatenate(
        [ps.reshape(BB, NS, D), pa.reshape(BB, NA, D)], axis=1)
    contrib = jnp.concatenate([
        jnp.sum(ps, axis=0, keepdims=True),
        jnp.sum(ps * ps, axis=0, keepdims=True),
        jnp.sum(pa, axis=0, keepdims=True),
        jnp.sum(pa * pa, axis=0, keepdims=True),
    ], axis=0)

    @pl.when(i == 0)
    def _():
        st_ref[...] = jnp.zeros_like(st_ref)

    st_ref[...] += contrib


def _affine(srow, sqrow, n, grow, brow):
    mu = srow * (1.0 / n)
    var = sqrow * (1.0 / n) - mu * mu
    a = grow * jax.lax.rsqrt(var + EPS)
    return a, brow - mu * a


def _attn_kernel(n_batch, p_ref, aff_ref, vp_ref, wqkv_ref,
                 mss_ref, msa_ref, mas_ref, maa_ref, z_ref, st_ref):
    i = pl.program_id(0)
    P = p_ref[...]
    D = P.shape[2]
    hd = D // H
    aff = aff_ref[...]
    a_s, c_s = _affine(aff[0:1], aff[1:2], n_batch * NS, aff[4:5], aff[5:6])
    a_a, c_a = _affine(aff[2:3], aff[3:4], n_batch * NA, aff[6:7], aff[7:8])
    vp = vp_ref[...]
    h_s = P[:, :NS, :] * a_s[None] + c_s[None] + vp[None, :NS, :]
    h_a = P[:, NS:, :] * a_a[None] + c_a[None] + vp[None, NS:, :]
    hbcat = jnp.concatenate([h_s, h_a], axis=1).astype(jnp.bfloat16)

    lane = jax.lax.broadcasted_iota(jnp.int32, (H, D), 1)
    head = jax.lax.broadcasted_iota(jnp.int32, (H, D), 0)
    hm = (lane // hd == head).astype(jnp.float32)

    masks = (mss_ref, msa_ref, mas_ref, maa_ref)
    z_s = jnp.zeros((BB, NS, D), dtype=jnp.float32)
    z_a = jnp.zeros((BB, NA, D), dtype=jnp.float32)
    scale = 1.0 / np.sqrt(hd)
    for e, ((s0, ns, d0, nd, k, dt), mref) in enumerate(zip(_EDGES, masks)):
        hb_src = hbcat[:, s0:s0 + ns, :]
        hb_dst = hbcat[:, d0:d0 + nd, :]
        wq = wqkv_ref[3 * e].astype(jnp.bfloat16)
        wk = wqkv_ref[3 * e + 1].astype(jnp.bfloat16)
        wv = wqkv_ref[3 * e + 2].astype(jnp.bfloat16)
        q = jax.lax.dot_general(hb_dst, wq, (((2,), (0,)), ((), ())),
                                preferred_element_type=jnp.float32)
        kk = jax.lax.dot_general(hb_src, wk, (((2,), (0,)), ((), ())),
                                 preferred_element_type=jnp.float32)
        vv = jax.lax.dot_general(hb_src, wv, (((2,), (0,)), ((), ())),
                                 preferred_element_type=jnp.float32)
        kst = (kk[:, None, :, :] * hm[None, :, None, :]).reshape(BB, H * ns, D)
        vst = (vv[:, None, :, :] * hm[None, :, None, :]).reshape(BB, H * ns, D)
        s4 = jax.lax.dot_general(kst, q, (((2,), (2,)), ((0,), (0,))),
                                 preferred_element_type=jnp.float32, precision=jax.lax.Precision.HIGHEST)
        s4 = (s4 * scale).reshape(BB, H, ns, nd)
        mask = mref[...]
        sm = s4 + ((mask - 1.0) * (-NEG))[None, None]
        m = jnp.max(sm, axis=2, keepdims=True)
        ex = jnp.exp(sm - m) * mask[None, None]
        denom = jnp.sum(ex, axis=2, keepdims=True) + 1e-16
        attn = (ex / denom).reshape(BB, H * ns, nd)
        agg = jax.lax.dot_general(attn, vst, (((1,), (1,)), ((0,), (0,))),
                                  preferred_element_type=jnp.float32, precision=jax.lax.Precision.HIGHEST)
        if dt == 0:
            z_s = z_s + agg
        else:
            z_a = z_a + agg
    z = jnp.concatenate([z_s, z_a], axis=1)
    z_ref[...] = z
    z2 = z.reshape(BB * NN, D)
    contrib = jnp.concatenate([
        jnp.sum(z2, axis=0, keepdims=True),
        jnp.sum(z2 * z2, axis=0, keepdims=True),
    ], axis=0)

    @pl.when(i == 0)
    def _():
        st_ref[...] = jnp.zeros_like(st_ref)

    st_ref[...] += contrib


def _gate_kernel(n_rows, z_ref, aff_ref, vp_ref, p_ref, st_ref):
    i = pl.program_id(0)
    z = z_ref[...]
    D = z.shape[2]
    aff = aff_ref[...]
    a, c = _affine(aff[0:1], aff[1:2], n_rows, aff[2:3], aff[3:4])
    zr = jnp.maximum(z * a[None] + c[None], 0.0)
    p = zr * vp_ref[...][None]
    p_ref[...] = p
    p2 = p.reshape(BB * NN, D)
    contrib = jnp.concatenate([
        jnp.sum(p2, axis=0, keepdims=True),
        jnp.sum(p2 * p2, axis=0, keepdims=True),
    ], axis=0)

    @pl.when(i == 0)
    def _():
        st_ref[...] = jnp.zeros_like(st_ref)

    st_ref[...] += contrib


def _out_kernel(n_rows, p_ref, aff_ref,
                w0s_ref, w1s_ref, w2s_ref, w0a_ref, w1a_ref, w2a_ref,
                bs_ref, ba_ref, b2s_ref, b2a_ref, so_ref, ao_ref):
    p = p_ref[...]
    D = p.shape[2]
    aff = aff_ref[...]
    a, c = _affine(aff[0:1], aff[1:2], n_rows, aff[2:3], aff[3:4])
    pn = jnp.maximum(p * a[None] + c[None], 0.0)
    ps = pn[:, :NS, :].reshape(BB * NS, D)
    pa = pn[:, NS:, :].reshape(BB * NA, D)
    bf = jnp.bfloat16
    hs = jnp.maximum(jnp.dot(ps.astype(bf), w0s_ref[...].astype(bf),
                             preferred_element_type=jnp.float32) + bs_ref[0:1], 0.0)
    hs = jnp.maximum(jnp.dot(hs.astype(bf), w1s_ref[...].astype(bf),
                             preferred_element_type=jnp.float32) + bs_ref[1:2], 0.0)
    os_ = jnp.dot(hs.astype(bf), w2s_ref[...].astype(bf),
                  preferred_element_type=jnp.float32) + b2s_ref[...]
    so_ref[...] = os_.reshape(BB, NS, os_.shape[1])
    ha = jnp.maximum(jnp.dot(pa.astype(bf), w0a_ref[...].astype(bf),
                             preferred_element_type=jnp.float32) + ba_ref[0:1], 0.0)
    ha = jnp.maximum(jnp.dot(ha.astype(bf), w1a_ref[...].astype(bf),
                             preferred_element_type=jnp.float32) + ba_ref[1:2], 0.0)
    oa = jnp.dot(ha.astype(bf), w2a_ref[...].astype(bf),
                 preferred_element_type=jnp.float32) + b2a_ref[...]
    ao_ref[...] = oa.reshape(BB, NA, oa.shape[1])


def _full(shape):
    return pl.BlockSpec(shape, lambda i: (0,) * len(shape))


def _batched(shape):
    return pl.BlockSpec(shape, lambda i: (i,) + (0,) * (len(shape) - 1))


def kernel(x, params):
    B, _, L = x.shape
    D = params["embed"].shape[1]
    nb = B // BB
    f32 = jnp.float32

    def row(v):
        return jnp.asarray(v, f32).reshape(1, -1)

    pvs = params["v_proj_sensor"]
    pva = params["v_proj_actuator"]
    vecs0 = jnp.concatenate([
        row(params["bn_emb"]["g"]), row(params["bn_emb"]["b"]),
        row(pvs["b"]), row(pvs["g"]), row(pvs["beta"]),
        row(pva["b"]), row(pva["g"]), row(pva["beta"]),
    ], axis=0)
    vp, m_ss, m_sa, m_as, m_aa = pl.pallas_call(
        _graph_kernel,
        out_shape=[
            jax.ShapeDtypeStruct((NN, D), f32),
            jax.ShapeDtypeStruct((NS, NS), f32),
            jax.ShapeDtypeStruct((NS, NA), f32),
            jax.ShapeDtypeStruct((NA, NS), f32),
            jax.ShapeDtypeStruct((NA, NA), f32),
        ],
    )(params["embed"], vecs0, pvs["W"], pva["W"])

    pxs = params["x_proj_sensor"]
    pxa = params["x_proj_actuator"]
    bvec = jnp.concatenate([row(pxs["b"]), row(pxa["b"])], axis=0)
    P, xstats = pl.pallas_call(
        _proj_kernel,
        grid=(nb,),
        in_specs=[_batched((BB, NN, L)), _full((L, D)), _full((L, D)),
                  _full((2, D))],
        out_specs=[_batched((BB, NN, D)), _full((4, D))],
        out_shape=[jax.ShapeDtypeStruct((B, NN, D), f32),
                   jax.ShapeDtypeStruct((4, D), f32)],
    )(x, pxs["W"], pxa["W"], bvec)

    aff1 = jnp.concatenate([
        xstats, row(pxs["g"]), row(pxs["beta"]), row(pxa["g"]), row(pxa["beta"]),
    ], axis=0)
    et_names = ["sensor__to__sensor", "sensor__to__actuator",
                "actuator__to__sensor", "actuator__to__actuator"]
    wqkv = jnp.stack([params["gl_" + n][w] for n in et_names
                      for w in ("Wq", "Wk", "Wv")], axis=0)
    z, zstats = pl.pallas_call(
        functools.partial(_attn_kernel, B),
        grid=(nb,),
        in_specs=[_batched((BB, NN, D)), _full((8, D)), _full((NN, D)),
                  _full((12, D, D)), _full((NS, NS)), _full((NS, NA)),
                  _full((NA, NS)), _full((NA, NA))],
        out_specs=[_batched((BB, NN, D)), _full((2, D))],
        out_shape=[jax.ShapeDtypeStruct((B, NN, D), f32),
                   jax.ShapeDtypeStruct((2, D), f32)],
    )(P, aff1, vp, wqkv, m_ss, m_sa, m_as, m_aa)

    aff2 = jnp.concatenate([zstats, row(params["bn_g"]["g"]),
                            row(params["bn_g"]["b"])], axis=0)
    Pg, pstats = pl.pallas_call(
        functools.partial(_gate_kernel, B * NN),
        grid=(nb,),
        in_specs=[_batched((BB, NN, D)), _full((4, D)), _full((NN, D))],
        out_specs=[_batched((BB, NN, D)), _full((2, D))],
        out_shape=[jax.ShapeDtypeStruct((B, NN, D), f32),
                   jax.ShapeDtypeStruct((2, D), f32)],
    )(z, aff2, vp)

    aff3 = jnp.concatenate([pstats, row(params["bn_p"]["g"]),
                            row(params["bn_p"]["b"])], axis=0)
    ms = params["sensor_out"]
    ma = params["actuator_out"]
    DOUT = ms["W0"].shape[1]
    d_s = ms["W2"].shape[1]
    d_a = ma["W2"].shape[1]
    bs = jnp.concatenate([row(ms["b0"]), row(ms["b1"])], axis=0)
    ba = jnp.concatenate([row(ma["b0"]), row(ma["b1"])], axis=0)
    sensor_out, actuator_out = pl.pallas_call(
        functools.partial(_out_kernel, B * NN),
        grid=(nb,),
        in_specs=[_batched((BB, NN, D)), _full((4, D)),
                  _full((D, DOUT)), _full((DOUT, DOUT)), _full((DOUT, d_s)),
                  _full((D, DOUT)), _full((DOUT, DOUT)), _full((DOUT, d_a)),
                  _full((2, DOUT)), _full((2, DOUT)),
                  _full((1, d_s)), _full((1, d_a))],
        out_specs=[_batched((BB, NS, d_s)), _batched((BB, NA, d_a))],
        out_shape=[jax.ShapeDtypeStruct((B, NS, d_s), f32),
                   jax.ShapeDtypeStruct((B, NA, d_a), f32)],
    )(Pg, aff3, ms["W0"], ms["W1"], ms["W2"], ma["W0"], ma["W1"], ma["W2"],
      bs, ba, row(ms["b2"]), row(ma["b2"]))

    return (sensor_out, actuator_out)


# speedup vs baseline: 544.9394x; 1.2464x over previous
import functools

import jax
import jax.numpy as jnp
import numpy as np
from jax.experimental import pallas as pl
from jax.experimental.pallas import tpu as pltpu

NS = 64
NA = 32
NN = 96
H = 4
EPS = 1e-5
NEG = -1e30

_EDGES = (
    (0, NS, 0, NS, 16, 0),
    (0, NS, NS, NA, 8, 1),
    (NS, NA, 0, NS, 8, 0),
    (NS, NA, NS, NA, 8, 1),
)

BB = 32


def _bn_rows(h, g, beta):
    n = h.shape[0]
    mu = jnp.sum(h, axis=0, keepdims=True) / n
    d = h - mu
    var = jnp.sum(d * d, axis=0, keepdims=True) / n
    return d / jnp.sqrt(var + EPS) * g + beta


def _topk_mask(sim, k):
    n_d = sim.shape[1]
    iota = jax.lax.broadcasted_iota(jnp.int32, sim.shape, 1)
    work = sim
    sel = jnp.zeros_like(sim)
    for _ in range(k):
        cm = jnp.max(work, axis=1, keepdims=True)
        cand = jnp.where(work == cm, iota, n_d)
        fi = jnp.min(cand, axis=1, keepdims=True)
        ch = iota == fi
        sel = jnp.maximum(sel, ch.astype(jnp.float32))
        work = jnp.where(ch, -3.0, work)
    return sel


def _graph_kernel(emb_ref, vecs_ref, wvs_ref, wva_ref,
                  vp_ref, mss_ref, msa_ref, mas_ref, maa_ref):
    emb = emb_ref[...]
    vecs = vecs_ref[...]
    v = _bn_rows(emb, vecs[0:1], vecs[1:2])
    vb = v.astype(jnp.bfloat16)
    hs = jnp.dot(vb[0:NS], wvs_ref[...].astype(jnp.bfloat16),
                 preferred_element_type=jnp.float32) + vecs[2:3]
    vps = _bn_rows(hs, vecs[3:4], vecs[4:5])
    ha = jnp.dot(vb[NS:NN], wva_ref[...].astype(jnp.bfloat16),
                 preferred_element_type=jnp.float32) + vecs[5:6]
    vpa = _bn_rows(ha, vecs[6:7], vecs[7:8])
    vp_ref[...] = jnp.concatenate([vps, vpa], axis=0)

    nrm = jnp.sqrt(jnp.sum(v * v, axis=1, keepdims=True)) + 1e-8
    outs = (mss_ref, msa_ref, mas_ref, maa_ref)
    for (s0, ns, d0, nd, k, _), oref in zip(_EDGES, outs):
        g = jax.lax.dot_general(vb[s0:s0 + ns], vb[d0:d0 + nd],
                                (((1,), (1,)), ((), ())),
                                preferred_element_type=jnp.float32)
        outer = jax.lax.dot_general(nrm[s0:s0 + ns], nrm[d0:d0 + nd],
                                    (((1,), (1,)), ((), ())),
                                    preferred_element_type=jnp.float32,
                                    precision=jax.lax.Precision.HIGHEST)
        oref[...] = _topk_mask(g / outer, k)


def _proj_kernel(x_ref, ws_ref, wa_ref, bv_ref, p_ref, st_ref):
    i = pl.program_id(0)
    xb = x_ref[...]
    L = xb.shape[2]
    D = ws_ref.shape[1]
    xs = xb[:, :NS, :].reshape(BB * NS, L)
    xa = xb[:, NS:, :].reshape(BB * NA, L)
    ps = jnp.dot(xs.astype(jnp.bfloat16), ws_ref[...].astype(jnp.bfloat16),
                 preferred_element_type=jnp.float32) + bv_ref[0:1]
    pa = jnp.dot(xa.astype(jnp.bfloat16), wa_ref[...].astype(jnp.bfloat16),
                 preferred_element_type=jnp.float32) + bv_ref[1:2]
    p_ref[...] = jnp.concatenate(
        [ps.reshape(BB, NS, D), pa.reshape(BB, NA, D)], axis=1)
    contrib = jnp.concatenate([
        jnp.sum(ps, axis=0, keepdims=True),
        jnp.sum(ps * ps, axis=0, keepdims=True),
        jnp.sum(pa, axis=0, keepdims=True),
        jnp.sum(pa * pa, axis=0, keepdims=True),
    ], axis=0)

    @pl.when(i == 0)
    def _():
        st_ref[...] = jnp.zeros_like(st_ref)

    st_ref[...] += contrib


def _dot3(a, b, dims):
    a1 = a.astype(jnp.bfloat16)
    a2 = (a - a1.astype(jnp.float32)).astype(jnp.bfloat16)
    b1 = b.astype(jnp.bfloat16)
    b2 = (b - b1.astype(jnp.float32)).astype(jnp.bfloat16)
    d = functools.partial(jax.lax.dot_general, dimension_numbers=dims,
                          preferred_element_type=jnp.float32)
    return d(a1, b1) + d(a1, b2) + d(a2, b1)


def _affine(srow, sqrow, n, grow, brow):
    mu = srow * (1.0 / n)
    var = sqrow * (1.0 / n) - mu * mu
    a = grow * jax.lax.rsqrt(var + EPS)
    return a, brow - mu * a


def _attn_kernel(n_batch, p_ref, aff_ref, vp_ref, wqkv_ref,
                 mss_ref, msa_ref, mas_ref, maa_ref, z_ref, st_ref):
    i = pl.program_id(0)
    P = p_ref[...]
    D = P.shape[2]
    hd = D // H
    aff = aff_ref[...]
    a_s, c_s = _affine(aff[0:1], aff[1:2], n_batch * NS, aff[4:5], aff[5:6])
    a_a, c_a = _affine(aff[2:3], aff[3:4], n_batch * NA, aff[6:7], aff[7:8])
    vp = vp_ref[...]
    h_s = P[:, :NS, :] * a_s[None] + c_s[None] + vp[None, :NS, :]
    h_a = P[:, NS:, :] * a_a[None] + c_a[None] + vp[None, NS:, :]
    hbcat = jnp.concatenate([h_s, h_a], axis=1).astype(jnp.bfloat16)

    lane = jax.lax.broadcasted_iota(jnp.int32, (H, D), 1)
    head = jax.lax.broadcasted_iota(jnp.int32, (H, D), 0)
    hm = (lane // hd == head).astype(jnp.float32)

    masks = (mss_ref, msa_ref, mas_ref, maa_ref)
    z_s = jnp.zeros((BB, NS, D), dtype=jnp.float32)
    z_a = jnp.zeros((BB, NA, D), dtype=jnp.float32)
    scale = 1.0 / np.sqrt(hd)
    hmb = hm.astype(jnp.bfloat16)
    bf = jnp.bfloat16
    f32 = jnp.float32
    for e, ((s0, ns, d0, nd, k, dt), mref) in enumerate(zip(_EDGES, masks)):
        hb_src = hbcat[:, s0:s0 + ns, :]
        hb_dst = hbcat[:, d0:d0 + nd, :]
        wq = wqkv_ref[3 * e].astype(bf)
        wk = wqkv_ref[3 * e + 1].astype(bf)
        wv = wqkv_ref[3 * e + 2].astype(bf)
        q = jax.lax.dot_general(hb_dst, wq, (((2,), (0,)), ((), ())),
                                preferred_element_type=f32) * scale
        kk = jax.lax.dot_general(hb_src, wk, (((2,), (0,)), ((), ())),
                                 preferred_element_type=f32)
        vv = jax.lax.dot_general(hb_src, wv, (((2,), (0,)), ((), ())),
                                 preferred_element_type=f32)
        q1 = q.astype(bf)
        q2 = (q - q1.astype(f32)).astype(bf)
        k1 = kk.astype(bf)
        k2 = (kk - k1.astype(f32)).astype(bf)
        v1 = vv.astype(bf)
        v2 = (vv - v1.astype(f32)).astype(bf)
        kst1 = (k1[:, None, :, :] * hmb[None, :, None, :]).reshape(BB, H * ns, D)
        kst2 = (k2[:, None, :, :] * hmb[None, :, None, :]).reshape(BB, H * ns, D)
        dq = functools.partial(jax.lax.dot_general,
                               dimension_numbers=(((2,), (2,)), ((0,), (0,))),
                               preferred_element_type=f32)
        s4 = (dq(kst1, q1) + dq(kst1, q2) + dq(kst2, q1)).reshape(BB, H, ns, nd)
        mask = mref[...]
        sm = s4 + ((mask - 1.0) * (-NEG))[None, None]
        m = jnp.max(sm, axis=2, keepdims=True)
        ex = jnp.exp(sm - m)
        he = jnp.max(mask, axis=0, keepdims=True)
        denom = jnp.sum(ex, axis=2, keepdims=True) + 1e-16
        rd = he[None, None] / denom
        attn = (ex * rd).reshape(BB, H * ns, nd)
        a1 = attn.astype(bf)
        a2 = (attn - a1.astype(f32)).astype(bf)
        vst1 = (v1[:, None, :, :] * hmb[None, :, None, :]).reshape(BB, H * ns, D)
        vst2 = (v2[:, None, :, :] * hmb[None, :, None, :]).reshape(BB, H * ns, D)
        da = functools.partial(jax.lax.dot_general,
                               dimension_numbers=(((1,), (1,)), ((0,), (0,))),
                               preferred_element_type=f32)
        agg = da(a1, vst1) + da(a1, vst2) + da(a2, vst1)
        if dt == 0:
            z_s = z_s + agg
        else:
            z_a = z_a + agg
    z = jnp.concatenate([z_s, z_a], axis=1)
    z_ref[...] = z
    z2 = z.reshape(BB * NN, D)
    contrib = jnp.concatenate([
        jnp.sum(z2, axis=0, keepdims=True),
        jnp.sum(z2 * z2, axis=0, keepdims=True),
    ], axis=0)

    @pl.when(i == 0)
    def _():
        st_ref[...] = jnp.zeros_like(st_ref)

    st_ref[...] += contrib


def _gate_kernel(n_rows, z_ref, aff_ref, vp_ref, p_ref, st_ref):
    i = pl.program_id(0)
    z = z_ref[...]
    D = z.shape[2]
    aff = aff_ref[...]
    a, c = _affine(aff[0:1], aff[1:2], n_rows, aff[2:3], aff[3:4])
    zr = jnp.maximum(z * a[None] + c[None], 0.0)
    p = zr * vp_ref[...][None]
    p_ref[...] = p
    p2 = p.reshape(BB * NN, D)
    contrib = jnp.concatenate([
        jnp.sum(p2, axis=0, keepdims=True),
        jnp.sum(p2 * p2, axis=0, keepdims=True),
    ], axis=0)

    @pl.when(i == 0)
    def _():
        st_ref[...] = jnp.zeros_like(st_ref)

    st_ref[...] += contrib


def _out_kernel(n_rows, p_ref, aff_ref,
                w0s_ref, w1s_ref, w2s_ref, w0a_ref, w1a_ref, w2a_ref,
                bs_ref, ba_ref, b2s_ref, b2a_ref, so_ref, ao_ref):
    p = p_ref[...]
    D = p.shape[2]
    aff = aff_ref[...]
    a, c = _affine(aff[0:1], aff[1:2], n_rows, aff[2:3], aff[3:4])
    pn = jnp.maximum(p * a[None] + c[None], 0.0)
    ps = pn[:, :NS, :].reshape(BB * NS, D)
    pa = pn[:, NS:, :].reshape(BB * NA, D)
    bf = jnp.bfloat16
    hs = jnp.maximum(jnp.dot(ps.astype(bf), w0s_ref[...].astype(bf),
                             preferred_element_type=jnp.float32) + bs_ref[0:1], 0.0)
    hs = jnp.maximum(jnp.dot(hs.astype(bf), w1s_ref[...].astype(bf),
                             preferred_element_type=jnp.float32) + bs_ref[1:2], 0.0)
    os_ = jnp.dot(hs.astype(bf), w2s_ref[...].astype(bf),
                  preferred_element_type=jnp.float32) + b2s_ref[...]
    so_ref[...] = os_.reshape(BB, NS, os_.shape[1])
    ha = jnp.maximum(jnp.dot(pa.astype(bf), w0a_ref[...].astype(bf),
                             preferred_element_type=jnp.float32) + ba_ref[0:1], 0.0)
    ha = jnp.maximum(jnp.dot(ha.astype(bf), w1a_ref[...].astype(bf),
                             preferred_element_type=jnp.float32) + ba_ref[1:2], 0.0)
    oa = jnp.dot(ha.astype(bf), w2a_ref[...].astype(bf),
                 preferred_element_type=jnp.float32) + b2a_ref[...]
    ao_ref[...] = oa.reshape(BB, NA, oa.shape[1])


def _full(shape):
    return pl.BlockSpec(shape, lambda i: (0,) * len(shape))


def _batched(shape):
    return pl.BlockSpec(shape, lambda i: (i,) + (0,) * (len(shape) - 1))


def kernel(x, params):
    B, _, L = x.shape
    D = params["embed"].shape[1]
    nb = B // BB
    f32 = jnp.float32

    def row(v):
        return jnp.asarray(v, f32).reshape(1, -1)

    pvs = params["v_proj_sensor"]
    pva = params["v_proj_actuator"]
    vecs0 = jnp.concatenate([
        row(params["bn_emb"]["g"]), row(params["bn_emb"]["b"]),
        row(pvs["b"]), row(pvs["g"]), row(pvs["beta"]),
        row(pva["b"]), row(pva["g"]), row(pva["beta"]),
    ], axis=0)
    vp, m_ss, m_sa, m_as, m_aa = pl.pallas_call(
        _graph_kernel,
        out_shape=[
            jax.ShapeDtypeStruct((NN, D), f32),
            jax.ShapeDtypeStruct((NS, NS), f32),
            jax.ShapeDtypeStruct((NS, NA), f32),
            jax.ShapeDtypeStruct((NA, NS), f32),
            jax.ShapeDtypeStruct((NA, NA), f32),
        ],
    )(params["embed"], vecs0, pvs["W"], pva["W"])

    pxs = params["x_proj_sensor"]
    pxa = params["x_proj_actuator"]
    bvec = jnp.concatenate([row(pxs["b"]), row(pxa["b"])], axis=0)
    P, xstats = pl.pallas_call(
        _proj_kernel,
        grid=(nb,),
        in_specs=[_batched((BB, NN, L)), _full((L, D)), _full((L, D)),
                  _full((2, D))],
        out_specs=[_batched((BB, NN, D)), _full((4, D))],
        out_shape=[jax.ShapeDtypeStruct((B, NN, D), f32),
                   jax.ShapeDtypeStruct((4, D), f32)],
    )(x, pxs["W"], pxa["W"], bvec)

    aff1 = jnp.concatenate([
        xstats, row(pxs["g"]), row(pxs["beta"]), row(pxa["g"]), row(pxa["beta"]),
    ], axis=0)
    et_names = ["sensor__to__sensor", "sensor__to__actuator",
                "actuator__to__sensor", "actuator__to__actuator"]
    wqkv = jnp.stack([params["gl_" + n][w] for n in et_names
                      for w in ("Wq", "Wk", "Wv")], axis=0)
    z, zstats = pl.pallas_call(
        functools.partial(_attn_kernel, B),
        grid=(nb,),
        in_specs=[_batched((BB, NN, D)), _full((8, D)), _full((NN, D)),
                  _full((12, D, D)), _full((NS, NS)), _full((NS, NA)),
                  _full((NA, NS)), _full((NA, NA))],
        out_specs=[_batched((BB, NN, D)), _full((2, D))],
        out_shape=[jax.ShapeDtypeStruct((B, NN, D), f32),
                   jax.ShapeDtypeStruct((2, D), f32)],
    )(P, aff1, vp, wqkv, m_ss, m_sa, m_as, m_aa)

    aff2 = jnp.concatenate([zstats, row(params["bn_g"]["g"]),
                            row(params["bn_g"]["b"])], axis=0)
    Pg, pstats = pl.pallas_call(
        functools.partial(_gate_kernel, B * NN),
        grid=(nb,),
        in_specs=[_batched((BB, NN, D)), _full((4, D)), _full((NN, D))],
        out_specs=[_batched((BB, NN, D)), _full((2, D))],
        out_shape=[jax.ShapeDtypeStruct((B, NN, D), f32),
                   jax.ShapeDtypeStruct((2, D), f32)],
    )(z, aff2, vp)

    aff3 = jnp.concatenate([pstats, row(params["bn_p"]["g"]),
                            row(params["bn_p"]["b"])], axis=0)
    ms = params["sensor_out"]
    ma = params["actuator_out"]
    DOUT = ms["W0"].shape[1]
    d_s = ms["W2"].shape[1]
    d_a = ma["W2"].shape[1]
    bs = jnp.concatenate([row(ms["b0"]), row(ms["b1"])], axis=0)
    ba = jnp.concatenate([row(ma["b0"]), row(ma["b1"])], axis=0)
    sensor_out, actuator_out = pl.pallas_call(
        functools.partial(_out_kernel, B * NN),
        grid=(nb,),
        in_specs=[_batched((BB, NN, D)), _full((4, D)),
                  _full((D, DOUT)), _full((DOUT, DOUT)), _full((DOUT, d_s)),
                  _full((D, DOUT)), _full((DOUT, DOUT)), _full((DOUT, d_a)),
                  _full((2, DOUT)), _full((2, DOUT)),
                  _full((1, d_s)), _full((1, d_a))],
        out_specs=[_batched((BB, NS, d_s)), _batched((BB, NA, d_a))],
        out_shape=[jax.ShapeDtypeStruct((B, NS, d_s), f32),
                   jax.ShapeDtypeStruct((B, NA, d_a), f32)],
    )(Pg, aff3, ms["W0"], ms["W1"], ms["W2"], ma["W0"], ma["W1"], ma["W2"],
      bs, ba, row(ms["b2"]), row(ma["b2"]))

    return (sensor_out, actuator_out)


# speedup vs baseline: 576.7580x; 1.0584x over previous
import functools

import jax
import jax.numpy as jnp
import numpy as np
from jax.experimental import pallas as pl
from jax.experimental.pallas import tpu as pltpu

NS = 64
NA = 32
NN = 96
H = 4
EPS = 1e-5
NEG = -1e30

_EDGES = (
    (0, NS, 0, NS, 16, 0),
    (0, NS, NS, NA, 8, 1),
    (NS, NA, 0, NS, 8, 0),
    (NS, NA, NS, NA, 8, 1),
)

BB = 64


def _bn_rows(h, g, beta):
    n = h.shape[0]
    mu = jnp.sum(h, axis=0, keepdims=True) / n
    d = h - mu
    var = jnp.sum(d * d, axis=0, keepdims=True) / n
    return d / jnp.sqrt(var + EPS) * g + beta


def _topk_mask(sim, k):
    n_d = sim.shape[1]
    iota = jax.lax.broadcasted_iota(jnp.int32, sim.shape, 1)
    work = sim
    sel = jnp.zeros_like(sim)
    for _ in range(k):
        cm = jnp.max(work, axis=1, keepdims=True)
        cand = jnp.where(work == cm, iota, n_d)
        fi = jnp.min(cand, axis=1, keepdims=True)
        ch = iota == fi
        sel = jnp.maximum(sel, ch.astype(jnp.float32))
        work = jnp.where(ch, -3.0, work)
    return sel


def _graph_kernel(emb_ref, vecs_ref, wvs_ref, wva_ref,
                  vp_ref, mss_ref, msa_ref, mas_ref, maa_ref):
    emb = emb_ref[...]
    vecs = vecs_ref[...]
    v = _bn_rows(emb, vecs[0:1], vecs[1:2])
    vb = v.astype(jnp.bfloat16)
    hs = jnp.dot(vb[0:NS], wvs_ref[...].astype(jnp.bfloat16),
                 preferred_element_type=jnp.float32) + vecs[2:3]
    vps = _bn_rows(hs, vecs[3:4], vecs[4:5])
    ha = jnp.dot(vb[NS:NN], wva_ref[...].astype(jnp.bfloat16),
                 preferred_element_type=jnp.float32) + vecs[5:6]
    vpa = _bn_rows(ha, vecs[6:7], vecs[7:8])
    vp_ref[...] = jnp.concatenate([vps, vpa], axis=0)

    nrm = jnp.sqrt(jnp.sum(v * v, axis=1, keepdims=True)) + 1e-8
    outs = (mss_ref, msa_ref, mas_ref, maa_ref)
    for (s0, ns, d0, nd, k, _), oref in zip(_EDGES, outs):
        g = jax.lax.dot_general(vb[s0:s0 + ns], vb[d0:d0 + nd],
                                (((1,), (1,)), ((), ())),
                                preferred_element_type=jnp.float32)
        outer = jax.lax.dot_general(nrm[s0:s0 + ns], nrm[d0:d0 + nd],
                                    (((1,), (1,)), ((), ())),
                                    preferred_element_type=jnp.float32,
                                    precision=jax.lax.Precision.HIGHEST)
        oref[...] = _topk_mask(g / outer, k)


def _proj_kernel(x_ref, ws_ref, wa_ref, bv_ref, p_ref, st_ref):
    i = pl.program_id(0)
    xb = x_ref[...]
    L = xb.shape[2]
    D = ws_ref.shape[1]
    xs = xb[:, :NS, :].reshape(BB * NS, L)
    xa = xb[:, NS:, :].reshape(BB * NA, L)
    ps = jnp.dot(xs.astype(jnp.bfloat16), ws_ref[...].astype(jnp.bfloat16),
                 preferred_element_type=jnp.float32) + bv_ref[0:1]
    pa = jnp.dot(xa.astype(jnp.bfloat16), wa_ref[...].astype(jnp.bfloat16),
                 preferred_element_type=jnp.float32) + bv_ref[1:2]
    p_ref[...] = jnp.concatenate(
        [ps.reshape(BB, NS, D), pa.reshape(BB, NA, D)], axis=1)
    contrib = jnp.concatenate([
        jnp.sum(ps, axis=0, keepdims=True),
        jnp.sum(ps * ps, axis=0, keepdims=True),
        jnp.sum(pa, axis=0, keepdims=True),
        jnp.sum(pa * pa, axis=0, keepdims=True),
    ], axis=0)

    @pl.when(i == 0)
    def _():
        st_ref[...] = jnp.zeros_like(st_ref)

    st_ref[...] += contrib


def _dot3(a, b, dims):
    a1 = a.astype(jnp.bfloat16)
    a2 = (a - a1.astype(jnp.float32)).astype(jnp.bfloat16)
    b1 = b.astype(jnp.bfloat16)
    b2 = (b - b1.astype(jnp.float32)).astype(jnp.bfloat16)
    d = functools.partial(jax.lax.dot_general, dimension_numbers=dims,
                          preferred_element_type=jnp.float32)
    return d(a1, b1) + d(a1, b2) + d(a2, b1)


def _affine(srow, sqrow, n, grow, brow):
    mu = srow * (1.0 / n)
    var = sqrow * (1.0 / n) - mu * mu
    a = grow * jax.lax.rsqrt(var + EPS)
    return a, brow - mu * a


def _attn_kernel(n_batch, p_ref, aff_ref, vp_ref, wqkv_ref,
                 mss_ref, msa_ref, mas_ref, maa_ref, z_ref, st_ref):
    i = pl.program_id(0)
    P = p_ref[...]
    D = P.shape[2]
    hd = D // H
    aff = aff_ref[...]
    a_s, c_s = _affine(aff[0:1], aff[1:2], n_batch * NS, aff[4:5], aff[5:6])
    a_a, c_a = _affine(aff[2:3], aff[3:4], n_batch * NA, aff[6:7], aff[7:8])
    vp = vp_ref[...]
    h_s = P[:, :NS, :] * a_s[None] + c_s[None] + vp[None, :NS, :]
    h_a = P[:, NS:, :] * a_a[None] + c_a[None] + vp[None, NS:, :]
    hbcat = jnp.concatenate([h_s, h_a], axis=1).astype(jnp.bfloat16)

    lane = jax.lax.broadcasted_iota(jnp.int32, (H, D), 1)
    head = jax.lax.broadcasted_iota(jnp.int32, (H, D), 0)
    hm = (lane // hd == head).astype(jnp.float32)

    masks = (mss_ref, msa_ref, mas_ref, maa_ref)
    z_s = jnp.zeros((BB, NS, D), dtype=jnp.float32)
    z_a = jnp.zeros((BB, NA, D), dtype=jnp.float32)
    scale = 1.0 / np.sqrt(hd)
    hmb = hm.astype(jnp.bfloat16)
    bf = jnp.bfloat16
    f32 = jnp.float32
    for e, ((s0, ns, d0, nd, k, dt), mref) in enumerate(zip(_EDGES, masks)):
        hb_src = hbcat[:, s0:s0 + ns, :]
        hb_dst = hbcat[:, d0:d0 + nd, :]
        wq = wqkv_ref[3 * e].astype(bf)
        wk = wqkv_ref[3 * e + 1].astype(bf)
        wv = wqkv_ref[3 * e + 2].astype(bf)
        q = jax.lax.dot_general(hb_dst, wq, (((2,), (0,)), ((), ())),
                                preferred_element_type=f32) * scale
        kk = jax.lax.dot_general(hb_src, wk, (((2,), (0,)), ((), ())),
                                 preferred_element_type=f32)
        vv = jax.lax.dot_general(hb_src, wv, (((2,), (0,)), ((), ())),
                                 preferred_element_type=f32)
        q1 = q.astype(bf)
        q2 = (q - q1.astype(f32)).astype(bf)
        k1 = kk.astype(bf)
        k2 = (kk - k1.astype(f32)).astype(bf)
        v1 = vv.astype(bf)
        v2 = (vv - v1.astype(f32)).astype(bf)
        kst1 = (k1[:, None, :, :] * hmb[None, :, None, :]).reshape(BB, H * ns, D)
        kst2 = (k2[:, None, :, :] * hmb[None, :, None, :]).reshape(BB, H * ns, D)
        dq = functools.partial(jax.lax.dot_general,
                               dimension_numbers=(((2,), (2,)), ((0,), (0,))),
                               preferred_element_type=f32)
        s4 = (dq(kst1, q1) + dq(kst1, q2) + dq(kst2, q1)).reshape(BB, H, ns, nd)
        mask = mref[...]
        sm = s4 + ((mask - 1.0) * (-NEG))[None, None]
        m = jnp.max(sm, axis=2, keepdims=True)
        ex = jnp.exp(sm - m)
        he = jnp.max(mask, axis=0, keepdims=True)
        denom = jnp.sum(ex, axis=2, keepdims=True) + 1e-16
        rd = he[None, None] / denom
        attn = (ex * rd).reshape(BB, H * ns, nd)
        a1 = attn.astype(bf)
        a2 = (attn - a1.astype(f32)).astype(bf)
        vst1 = (v1[:, None, :, :] * hmb[None, :, None, :]).reshape(BB, H * ns, D)
        vst2 = (v2[:, None, :, :] * hmb[None, :, None, :]).reshape(BB, H * ns, D)
        da = functools.partial(jax.lax.dot_general,
                               dimension_numbers=(((1,), (1,)), ((0,), (0,))),
                               preferred_element_type=f32)
        agg = da(a1, vst1) + da(a1, vst2) + da(a2, vst1)
        if dt == 0:
            z_s = z_s + agg
        else:
            z_a = z_a + agg
    z = jnp.concatenate([z_s, z_a], axis=1)
    z_ref[...] = z
    z2 = z.reshape(BB * NN, D)
    contrib = jnp.concatenate([
        jnp.sum(z2, axis=0, keepdims=True),
        jnp.sum(z2 * z2, axis=0, keepdims=True),
    ], axis=0)

    @pl.when(i == 0)
    def _():
        st_ref[...] = jnp.zeros_like(st_ref)

    st_ref[...] += contrib


def _gate_kernel(n_rows, z_ref, aff_ref, vp_ref, p_ref, st_ref):
    i = pl.program_id(0)
    z = z_ref[...]
    D = z.shape[2]
    aff = aff_ref[...]
    a, c = _affine(aff[0:1], aff[1:2], n_rows, aff[2:3], aff[3:4])
    zr = jnp.maximum(z * a[None] + c[None], 0.0)
    p = zr * vp_ref[...][None]
    p_ref[...] = p
    p2 = p.reshape(BB * NN, D)
    contrib = jnp.concatenate([
        jnp.sum(p2, axis=0, keepdims=True),
        jnp.sum(p2 * p2, axis=0, keepdims=True),
    ], axis=0)

    @pl.when(i == 0)
    def _():
        st_ref[...] = jnp.zeros_like(st_ref)

    st_ref[...] += contrib


def _out_kernel(n_rows, p_ref, aff_ref,
                w0s_ref, w1s_ref, w2s_ref, w0a_ref, w1a_ref, w2a_ref,
                bs_ref, ba_ref, b2s_ref, b2a_ref, so_ref, ao_ref):
    p = p_ref[...]
    D = p.shape[2]
    aff = aff_ref[...]
    a, c = _affine(aff[0:1], aff[1:2], n_rows, aff[2:3], aff[3:4])
    pn = jnp.maximum(p * a[None] + c[None], 0.0)
    ps = pn[:, :NS, :].reshape(BB * NS, D)
    pa = pn[:, NS:, :].reshape(BB * NA, D)
    bf = jnp.bfloat16
    hs = jnp.maximum(jnp.dot(ps.astype(bf), w0s_ref[...].astype(bf),
                             preferred_element_type=jnp.float32) + bs_ref[0:1], 0.0)
    hs = jnp.maximum(jnp.dot(hs.astype(bf), w1s_ref[...].astype(bf),
                             preferred_element_type=jnp.float32) + bs_ref[1:2], 0.0)
    os_ = jnp.dot(hs.astype(bf), w2s_ref[...].astype(bf),
                  preferred_element_type=jnp.float32) + b2s_ref[...]
    so_ref[...] = os_.reshape(BB, NS, os_.shape[1])
    ha = jnp.maximum(jnp.dot(pa.astype(bf), w0a_ref[...].astype(bf),
                             preferred_element_type=jnp.float32) + ba_ref[0:1], 0.0)
    ha = jnp.maximum(jnp.dot(ha.astype(bf), w1a_ref[...].astype(bf),
                             preferred_element_type=jnp.float32) + ba_ref[1:2], 0.0)
    oa = jnp.dot(ha.astype(bf), w2a_ref[...].astype(bf),
                 preferred_element_type=jnp.float32) + b2a_ref[...]
    ao_ref[...] = oa.reshape(BB, NA, oa.shape[1])


def _full(shape):
    return pl.BlockSpec(shape, lambda i: (0,) * len(shape))


def _batched(shape):
    return pl.BlockSpec(shape, lambda i: (i,) + (0,) * (len(shape) - 1))


def kernel(x, params):
    B, _, L = x.shape
    D = params["embed"].shape[1]
    nb = B // BB
    f32 = jnp.float32

    def row(v):
        return jnp.asarray(v, f32).reshape(1, -1)

    pvs = params["v_proj_sensor"]
    pva = params["v_proj_actuator"]
    vecs0 = jnp.concatenate([
        row(params["bn_emb"]["g"]), row(params["bn_emb"]["b"]),
        row(pvs["b"]), row(pvs["g"]), row(pvs["beta"]),
        row(pva["b"]), row(pva["g"]), row(pva["beta"]),
    ], axis=0)
    vp, m_ss, m_sa, m_as, m_aa = pl.pallas_call(
        _graph_kernel,
        out_shape=[
            jax.ShapeDtypeStruct((NN, D), f32),
            jax.ShapeDtypeStruct((NS, NS), f32),
            jax.ShapeDtypeStruct((NS, NA), f32),
            jax.ShapeDtypeStruct((NA, NS), f32),
            jax.ShapeDtypeStruct((NA, NA), f32),
        ],
    )(params["embed"], vecs0, pvs["W"], pva["W"])

    pxs = params["x_proj_sensor"]
    pxa = params["x_proj_actuator"]
    bvec = jnp.concatenate([row(pxs["b"]), row(pxa["b"])], axis=0)
    P, xstats = pl.pallas_call(
        _proj_kernel,
        grid=(nb,),
        in_specs=[_batched((BB, NN, L)), _full((L, D)), _full((L, D)),
                  _full((2, D))],
        out_specs=[_batched((BB, NN, D)), _full((4, D))],
        out_shape=[jax.ShapeDtypeStruct((B, NN, D), f32),
                   jax.ShapeDtypeStruct((4, D), f32)],
    )(x, pxs["W"], pxa["W"], bvec)

    aff1 = jnp.concatenate([
        xstats, row(pxs["g"]), row(pxs["beta"]), row(pxa["g"]), row(pxa["beta"]),
    ], axis=0)
    et_names = ["sensor__to__sensor", "sensor__to__actuator",
                "actuator__to__sensor", "actuator__to__actuator"]
    wqkv = jnp.stack([params["gl_" + n][w] for n in et_names
                      for w in ("Wq", "Wk", "Wv")], axis=0)
    z, zstats = pl.pallas_call(
        functools.partial(_attn_kernel, B),
        grid=(nb,),
        in_specs=[_batched((BB, NN, D)), _full((8, D)), _full((NN, D)),
                  _full((12, D, D)), _full((NS, NS)), _full((NS, NA)),
                  _full((NA, NS)), _full((NA, NA))],
        out_specs=[_batched((BB, NN, D)), _full((2, D))],
        out_shape=[jax.ShapeDtypeStruct((B, NN, D), f32),
                   jax.ShapeDtypeStruct((2, D), f32)],
    )(P, aff1, vp, wqkv, m_ss, m_sa, m_as, m_aa)

    aff2 = jnp.concatenate([zstats, row(params["bn_g"]["g"]),
                            row(params["bn_g"]["b"])], axis=0)
    Pg, pstats = pl.pallas_call(
        functools.partial(_gate_kernel, B * NN),
        grid=(nb,),
        in_specs=[_batched((BB, NN, D)), _full((4, D)), _full((NN, D))],
        out_specs=[_batched((BB, NN, D)), _full((2, D))],
        out_shape=[jax.ShapeDtypeStruct((B, NN, D), f32),
                   jax.ShapeDtypeStruct((2, D), f32)],
    )(z, aff2, vp)

    aff3 = jnp.concatenate([pstats, row(params["bn_p"]["g"]),
                            row(params["bn_p"]["b"])], axis=0)
    ms = params["sensor_out"]
    ma = params["actuator_out"]
    DOUT = ms["W0"].shape[1]
    d_s = ms["W2"].shape[1]
    d_a = ma["W2"].shape[1]
    bs = jnp.concatenate([row(ms["b0"]), row(ms["b1"])], axis=0)
    ba = jnp.concatenate([row(ma["b0"]), row(ma["b1"])], axis=0)
    sensor_out, actuator_out = pl.pallas_call(
        functools.partial(_out_kernel, B * NN),
        grid=(nb,),
        in_specs=[_batched((BB, NN, D)), _full((4, D)),
                  _full((D, DOUT)), _full((DOUT, DOUT)), _full((DOUT, d_s)),
                  _full((D, DOUT)), _full((DOUT, DOUT)), _full((DOUT, d_a)),
                  _full((2, DOUT)), _full((2, DOUT)),
                  _full((1, d_s)), _full((1, d_a))],
        out_specs=[_batched((BB, NS, d_s)), _batched((BB, NA, d_a))],
        out_shape=[jax.ShapeDtypeStruct((B, NS, d_s), f32),
                   jax.ShapeDtypeStruct((B, NA, d_a), f32)],
    )(Pg, aff3, ms["W0"], ms["W1"], ms["W2"], ma["W0"], ma["W1"], ma["W2"],
      bs, ba, row(ms["b2"]), row(ma["b2"]))

    return (sensor_out, actuator_out)


# speedup vs baseline: 587.6549x; 1.0189x over previous
import functools

import jax
import jax.numpy as jnp
import numpy as np
from jax.experimental import pallas as pl
from jax.experimental.pallas import tpu as pltpu

NS = 64
NA = 32
NN = 96
H = 4
EPS = 1e-5
NEG = -1e30

_EDGES = (
    (0, NS, 0, NS, 16, 0),
    (0, NS, NS, NA, 8, 1),
    (NS, NA, 0, NS, 8, 0),
    (NS, NA, NS, NA, 8, 1),
)

BB = 64


def _bn_rows(h, g, beta):
    n = h.shape[0]
    mu = jnp.sum(h, axis=0, keepdims=True) / n
    d = h - mu
    var = jnp.sum(d * d, axis=0, keepdims=True) / n
    return d / jnp.sqrt(var + EPS) * g + beta


def _topk_mask(sim, k):
    n_d = sim.shape[1]
    iota = jax.lax.broadcasted_iota(jnp.int32, sim.shape, 1)
    work = sim
    sel = jnp.zeros_like(sim)
    for _ in range(k):
        cm = jnp.max(work, axis=1, keepdims=True)
        cand = jnp.where(work == cm, iota, n_d)
        fi = jnp.min(cand, axis=1, keepdims=True)
        ch = iota == fi
        sel = jnp.maximum(sel, ch.astype(jnp.float32))
        work = jnp.where(ch, -3.0, work)
    return sel


def _graph_build(emb_ref, vecs_ref, wvs_ref, wva_ref,
                 vp_ref, mss_ref, msa_ref, mas_ref, maa_ref):
    emb = emb_ref[...]
    vecs = vecs_ref[...]
    v = _bn_rows(emb, vecs[0:1], vecs[1:2])
    vb = v.astype(jnp.bfloat16)
    hs = jnp.dot(vb[0:NS], wvs_ref[...].astype(jnp.bfloat16),
                 preferred_element_type=jnp.float32) + vecs[2:3]
    vps = _bn_rows(hs, vecs[3:4], vecs[4:5])
    ha = jnp.dot(vb[NS:NN], wva_ref[...].astype(jnp.bfloat16),
                 preferred_element_type=jnp.float32) + vecs[5:6]
    vpa = _bn_rows(ha, vecs[6:7], vecs[7:8])
    vp_ref[...] = jnp.concatenate([vps, vpa], axis=0)

    nrm = jnp.sqrt(jnp.sum(v * v, axis=1, keepdims=True)) + 1e-8
    outs = (mss_ref, msa_ref, mas_ref, maa_ref)
    for (s0, ns, d0, nd, k, _), oref in zip(_EDGES, outs):
        g = jax.lax.dot_general(vb[s0:s0 + ns], vb[d0:d0 + nd],
                                (((1,), (1,)), ((), ())),
                                preferred_element_type=jnp.float32)
        outer = jax.lax.dot_general(nrm[s0:s0 + ns], nrm[d0:d0 + nd],
                                    (((1,), (1,)), ((), ())),
                                    preferred_element_type=jnp.float32,
                                    precision=jax.lax.Precision.HIGHEST)
        oref[...] = _topk_mask(g / outer, k)


def _proj_kernel(x_ref, ws_ref, wa_ref, bv_ref,
                 emb_ref, vecs_ref, wvs_ref, wva_ref,
                 p_ref, st_ref, vp_ref, mss_ref, msa_ref, mas_ref, maa_ref):
    i = pl.program_id(0)

    @pl.when(i == 0)
    def _():
        _graph_build(emb_ref, vecs_ref, wvs_ref, wva_ref,
                     vp_ref, mss_ref, msa_ref, mas_ref, maa_ref)

    xb = x_ref[...]
    L = xb.shape[2]
    D = ws_ref.shape[1]
    xs = xb[:, :NS, :].reshape(BB * NS, L)
    xa = xb[:, NS:, :].reshape(BB * NA, L)
    ps = jnp.dot(xs.astype(jnp.bfloat16), ws_ref[...].astype(jnp.bfloat16),
                 preferred_element_type=jnp.float32) + bv_ref[0:1]
    pa = jnp.dot(xa.astype(jnp.bfloat16), wa_ref[...].astype(jnp.bfloat16),
                 preferred_element_type=jnp.float32) + bv_ref[1:2]
    p_ref[...] = jnp.concatenate(
        [ps.reshape(BB, NS, D), pa.reshape(BB, NA, D)], axis=1)
    contrib = jnp.concatenate([
        jnp.sum(ps, axis=0, keepdims=True),
        jnp.sum(ps * ps, axis=0, keepdims=True),
        jnp.sum(pa, axis=0, keepdims=True),
        jnp.sum(pa * pa, axis=0, keepdims=True),
    ], axis=0)

    @pl.when(i == 0)
    def _():
        st_ref[...] = jnp.zeros_like(st_ref)

    st_ref[...] += contrib


def _dot3(a, b, dims):
    a1 = a.astype(jnp.bfloat16)
    a2 = (a - a1.astype(jnp.float32)).astype(jnp.bfloat16)
    b1 = b.astype(jnp.bfloat16)
    b2 = (b - b1.astype(jnp.float32)).astype(jnp.bfloat16)
    d = functools.partial(jax.lax.dot_general, dimension_numbers=dims,
                          preferred_element_type=jnp.float32)
    return d(a1, b1) + d(a1, b2) + d(a2, b1)


def _affine(srow, sqrow, n, grow, brow):
    mu = srow * (1.0 / n)
    var = sqrow * (1.0 / n) - mu * mu
    a = grow * jax.lax.rsqrt(var + EPS)
    return a, brow - mu * a


def _attn_kernel(n_batch, p_ref, aff_ref, vp_ref, wqkv_ref,
                 mss_ref, msa_ref, mas_ref, maa_ref, z_ref, st_ref):
    i = pl.program_id(0)
    P = p_ref[...]
    D = P.shape[2]
    hd = D // H
    aff = aff_ref[...]
    a_s, c_s = _affine(aff[0:1], aff[1:2], n_batch * NS, aff[4:5], aff[5:6])
    a_a, c_a = _affine(aff[2:3], aff[3:4], n_batch * NA, aff[6:7], aff[7:8])
    vp = vp_ref[...]
    h_s = P[:, :NS, :] * a_s[None] + c_s[None] + vp[None, :NS, :]
    h_a = P[:, NS:, :] * a_a[None] + c_a[None] + vp[None, NS:, :]
    hbcat = jnp.concatenate([h_s, h_a], axis=1).astype(jnp.bfloat16)

    lane = jax.lax.broadcasted_iota(jnp.int32, (H, D), 1)
    head = jax.lax.broadcasted_iota(jnp.int32, (H, D), 0)
    hm = (lane // hd == head).astype(jnp.float32)

    masks = (mss_ref, msa_ref, mas_ref, maa_ref)
    z_s = jnp.zeros((BB, NS, D), dtype=jnp.float32)
    z_a = jnp.zeros((BB, NA, D), dtype=jnp.float32)
    scale = 1.0 / np.sqrt(hd)
    hmb = hm.astype(jnp.bfloat16)
    bf = jnp.bfloat16
    f32 = jnp.float32
    for e, ((s0, ns, d0, nd, k, dt), mref) in enumerate(zip(_EDGES, masks)):
        hb_src = hbcat[:, s0:s0 + ns, :]
        hb_dst = hbcat[:, d0:d0 + nd, :]
        wq = wqkv_ref[3 * e].astype(bf)
        wk = wqkv_ref[3 * e + 1].astype(bf)
        wv = wqkv_ref[3 * e + 2].astype(bf)
        q = jax.lax.dot_general(hb_dst, wq, (((2,), (0,)), ((), ())),
                                preferred_element_type=f32) * scale
        kk = jax.lax.dot_general(hb_src, wk, (((2,), (0,)), ((), ())),
                                 preferred_element_type=f32)
        vv = jax.lax.dot_general(hb_src, wv, (((2,), (0,)), ((), ())),
                                 preferred_element_type=f32)
        q1 = q.astype(bf)
        q2 = (q - q1.astype(f32)).astype(bf)
        k1 = kk.astype(bf)
        k2 = (kk - k1.astype(f32)).astype(bf)
        v1 = vv.astype(bf)
        v2 = (vv - v1.astype(f32)).astype(bf)
        kst1 = (k1[:, None, :, :] * hmb[None, :, None, :]).reshape(BB, H * ns, D)
        kst2 = (k2[:, None, :, :] * hmb[None, :, None, :]).reshape(BB, H * ns, D)
        dq = functools.partial(jax.lax.dot_general,
                               dimension_numbers=(((2,), (2,)), ((0,), (0,))),
                               preferred_element_type=f32)
        s4 = (dq(kst1, q1) + dq(kst1, q2) + dq(kst2, q1)).reshape(BB, H, ns, nd)
        mask = mref[...]
        sm = s4 + ((mask - 1.0) * (-NEG))[None, None]
        m = jnp.max(sm, axis=2, keepdims=True)
        ex = jnp.exp(sm - m)
        he = jnp.max(mask, axis=0, keepdims=True)
        denom = jnp.sum(ex, axis=2, keepdims=True) + 1e-16
        rd = he[None, None] / denom
        attn = (ex * rd).reshape(BB, H * ns, nd)
        a1 = attn.astype(bf)
        a2 = (attn - a1.astype(f32)).astype(bf)
        vst1 = (v1[:, None, :, :] * hmb[None, :, None, :]).reshape(BB, H * ns, D)
        vst2 = (v2[:, None, :, :] * hmb[None, :, None, :]).reshape(BB, H * ns, D)
        da = functools.partial(jax.lax.dot_general,
                               dimension_numbers=(((1,), (1,)), ((0,), (0,))),
                               preferred_element_type=f32)
        agg = da(a1, vst1) + da(a1, vst2) + da(a2, vst1)
        if dt == 0:
            z_s = z_s + agg
        else:
            z_a = z_a + agg
    z = jnp.concatenate([z_s, z_a], axis=1)
    z_ref[...] = z
    z2 = z.reshape(BB * NN, D)
    contrib = jnp.concatenate([
        jnp.sum(z2, axis=0, keepdims=True),
        jnp.sum(z2 * z2, axis=0, keepdims=True),
    ], axis=0)

    @pl.when(i == 0)
    def _():
        st_ref[...] = jnp.zeros_like(st_ref)

    st_ref[...] += contrib


def _gate_kernel(n_rows, z_ref, aff_ref, vp_ref, st_ref):
    i = pl.program_id(0)
    z = z_ref[...]
    D = z.shape[2]
    aff = aff_ref[...]
    a, c = _affine(aff[0:1], aff[1:2], n_rows, aff[2:3], aff[3:4])
    zr = jnp.maximum(z * a[None] + c[None], 0.0)
    p = zr * vp_ref[...][None]
    p2 = p.reshape(BB * NN, D)
    contrib = jnp.concatenate([
        jnp.sum(p2, axis=0, keepdims=True),
        jnp.sum(p2 * p2, axis=0, keepdims=True),
    ], axis=0)

    @pl.when(i == 0)
    def _():
        st_ref[...] = jnp.zeros_like(st_ref)

    st_ref[...] += contrib


def _out_kernel(n_rows, z_ref, affg_ref, aff_ref, vp_ref,
                w0s_ref, w1s_ref, w2s_ref, w0a_ref, w1a_ref, w2a_ref,
                bs_ref, ba_ref, b2s_ref, b2a_ref, so_ref, ao_ref):
    z = z_ref[...]
    D = z.shape[2]
    affg = affg_ref[...]
    ag, cg = _affine(affg[0:1], affg[1:2], n_rows, affg[2:3], affg[3:4])
    p = jnp.maximum(z * ag[None] + cg[None], 0.0) * vp_ref[...][None]
    aff = aff_ref[...]
    a, c = _affine(aff[0:1], aff[1:2], n_rows, aff[2:3], aff[3:4])
    pn = jnp.maximum(p * a[None] + c[None], 0.0)
    ps = pn[:, :NS, :].reshape(BB * NS, D)
    pa = pn[:, NS:, :].reshape(BB * NA, D)
    bf = jnp.bfloat16
    hs = jnp.maximum(jnp.dot(ps.astype(bf), w0s_ref[...].astype(bf),
                             preferred_element_type=jnp.float32) + bs_ref[0:1], 0.0)
    hs = jnp.maximum(jnp.dot(hs.astype(bf), w1s_ref[...].astype(bf),
                             preferred_element_type=jnp.float32) + bs_ref[1:2], 0.0)
    os_ = jnp.dot(hs.astype(bf), w2s_ref[...].astype(bf),
                  preferred_element_type=jnp.float32) + b2s_ref[...]
    so_ref[...] = os_.reshape(BB, NS, os_.shape[1])
    ha = jnp.maximum(jnp.dot(pa.astype(bf), w0a_ref[...].astype(bf),
                             preferred_element_type=jnp.float32) + ba_ref[0:1], 0.0)
    ha = jnp.maximum(jnp.dot(ha.astype(bf), w1a_ref[...].astype(bf),
                             preferred_element_type=jnp.float32) + ba_ref[1:2], 0.0)
    oa = jnp.dot(ha.astype(bf), w2a_ref[...].astype(bf),
                 preferred_element_type=jnp.float32) + b2a_ref[...]
    ao_ref[...] = oa.reshape(BB, NA, oa.shape[1])


def _full(shape):
    return pl.BlockSpec(shape, lambda i: (0,) * len(shape))


def _batched(shape):
    return pl.BlockSpec(shape, lambda i: (i,) + (0,) * (len(shape) - 1))


def kernel(x, params):
    B, _, L = x.shape
    D = params["embed"].shape[1]
    nb = B // BB
    f32 = jnp.float32

    def row(v):
        return jnp.asarray(v, f32).reshape(1, -1)

    pvs = params["v_proj_sensor"]
    pva = params["v_proj_actuator"]
    vecs0 = jnp.concatenate([
        row(params["bn_emb"]["g"]), row(params["bn_emb"]["b"]),
        row(pvs["b"]), row(pvs["g"]), row(pvs["beta"]),
        row(pva["b"]), row(pva["g"]), row(pva["beta"]),
    ], axis=0)
    pxs = params["x_proj_sensor"]
    pxa = params["x_proj_actuator"]
    bvec = jnp.concatenate([row(pxs["b"]), row(pxa["b"])], axis=0)
    P, xstats, vp, m_ss, m_sa, m_as, m_aa = pl.pallas_call(
        _proj_kernel,
        grid=(nb,),
        in_specs=[_batched((BB, NN, L)), _full((L, D)), _full((L, D)),
                  _full((2, D)),
                  _full((NN, D)), _full((8, D)), _full((D, D)), _full((D, D))],
        out_specs=[_batched((BB, NN, D)), _full((4, D)),
                   _full((NN, D)), _full((NS, NS)), _full((NS, NA)),
                   _full((NA, NS)), _full((NA, NA))],
        out_shape=[jax.ShapeDtypeStruct((B, NN, D), f32),
                   jax.ShapeDtypeStruct((4, D), f32),
                   jax.ShapeDtypeStruct((NN, D), f32),
                   jax.ShapeDtypeStruct((NS, NS), f32),
                   jax.ShapeDtypeStruct((NS, NA), f32),
                   jax.ShapeDtypeStruct((NA, NS), f32),
                   jax.ShapeDtypeStruct((NA, NA), f32)],
    )(x, pxs["W"], pxa["W"], bvec,
      params["embed"], vecs0, pvs["W"], pva["W"])

    aff1 = jnp.concatenate([
        xstats, row(pxs["g"]), row(pxs["beta"]), row(pxa["g"]), row(pxa["beta"]),
    ], axis=0)
    et_names = ["sensor__to__sensor", "sensor__to__actuator",
                "actuator__to__sensor", "actuator__to__actuator"]
    wqkv = jnp.stack([params["gl_" + n][w] for n in et_names
                      for w in ("Wq", "Wk", "Wv")], axis=0)
    z, zstats = pl.pallas_call(
        functools.partial(_attn_kernel, B),
        grid=(nb,),
        in_specs=[_batched((BB, NN, D)), _full((8, D)), _full((NN, D)),
                  _full((12, D, D)), _full((NS, NS)), _full((NS, NA)),
                  _full((NA, NS)), _full((NA, NA))],
        out_specs=[_batched((BB, NN, D)), _full((2, D))],
        out_shape=[jax.ShapeDtypeStruct((B, NN, D), f32),
                   jax.ShapeDtypeStruct((2, D), f32)],
    )(P, aff1, vp, wqkv, m_ss, m_sa, m_as, m_aa)

    aff2 = jnp.concatenate([zstats, row(params["bn_g"]["g"]),
                            row(params["bn_g"]["b"])], axis=0)
    (pstats,) = pl.pallas_call(
        functools.partial(_gate_kernel, B * NN),
        grid=(nb,),
        in_specs=[_batched((BB, NN, D)), _full((4, D)), _full((NN, D))],
        out_specs=[_full((2, D))],
        out_shape=[jax.ShapeDtypeStruct((2, D), f32)],
    )(z, aff2, vp)

    aff3 = jnp.concatenate([pstats, row(params["bn_p"]["g"]),
                            row(params["bn_p"]["b"])], axis=0)
    ms = params["sensor_out"]
    ma = params["actuator_out"]
    DOUT = ms["W0"].shape[1]
    d_s = ms["W2"].shape[1]
    d_a = ma["W2"].shape[1]
    bs = jnp.concatenate([row(ms["b0"]), row(ms["b1"])], axis=0)
    ba = jnp.concatenate([row(ma["b0"]), row(ma["b1"])], axis=0)
    sensor_out, actuator_out = pl.pallas_call(
        functools.partial(_out_kernel, B * NN),
        grid=(nb,),
        in_specs=[_batched((BB, NN, D)), _full((4, D)), _full((4, D)),
                  _full((NN, D)),
                  _full((D, DOUT)), _full((DOUT, DOUT)), _full((DOUT, d_s)),
                  _full((D, DOUT)), _full((DOUT, DOUT)), _full((DOUT, d_a)),
                  _full((2, DOUT)), _full((2, DOUT)),
                  _full((1, d_s)), _full((1, d_a))],
        out_specs=[_batched((BB, NS, d_s)), _batched((BB, NA, d_a))],
        out_shape=[jax.ShapeDtypeStruct((B, NS, d_s), f32),
                   jax.ShapeDtypeStruct((B, NA, d_a), f32)],
    )(z, aff2, aff3, vp, ms["W0"], ms["W1"], ms["W2"], ma["W0"], ma["W1"], ma["W2"],
      bs, ba, row(ms["b2"]), row(ma["b2"]))

    return (sensor_out, actuator_out)
